# Initial kernel scaffold; baseline (speedup 1.0000x reference)
import jax
import jax.numpy as jnp
from jax.experimental import pallas as pl


def kernel(x, y_one_hot_train, W, sigmas, row, col):
    raise NotImplementedError("write your pallas kernel here")


# speedup vs baseline: 4.3657x; 4.3657x over previous
import functools

import jax
import jax.numpy as jnp
from jax import lax
from jax.experimental import pallas as pl
from jax.experimental.pallas import tpu as pltpu
from jax.experimental.pallas import tpu_sc as plsc

NC = 2
NS = 16
L = 16


def _make_step(n, e, d, block_e=80, chunk_rows=80):
    ept = e // (NC * NS)
    nblk = ept // block_e
    assert ept % block_e == 0
    assert n % chunk_rows == 0 and chunk_rows % 8 == 0
    nchunk = n // chunk_rows
    chunk_iters = -(-nchunk // NS)
    kd = d // L
    npad = -(-n // 1024) * 1024
    nslot = npad // 8
    dpt = nslot // NS

    mesh = plsc.VectorSubcoreMesh(
        core_axis_name="c", subcore_axis_name="s", num_cores=NC, num_subcores=NS
    )

    @functools.partial(
        pl.kernel,
        out_type=(
            jax.ShapeDtypeStruct((NC, n, d), jnp.float32),
            jax.ShapeDtypeStruct((NC, nslot, 128), jnp.float32),
        ),
        mesh=mesh,
        scratch_types=[
            pltpu.VMEM((block_e,), jnp.int32),
            pltpu.VMEM((block_e,), jnp.int32),
            pltpu.VMEM((block_e,), jnp.int32),
            pltpu.VMEM((block_e, d), jnp.float32),
            pltpu.VMEM((block_e, d), jnp.float32),
            pltpu.VMEM((block_e, 128), jnp.float32),
            pltpu.VMEM((128,), jnp.float32),
            pltpu.VMEM_SHARED((n, d), jnp.float32),
            pltpu.VMEM_SHARED((nslot, 128), jnp.float32),
            pltpu.SemaphoreType.DMA,
            pltpu.SemaphoreType.DMA,
        ],
    )
    def step(x_hbm, row_hbm, col_hbm, sig_hbm, s_out, deg_out,
             ridx_v, cidx_v, didx_v, rrows_v, crows_v, vbuf_v,
             sig_v, s_sh, deg_sh, sem1, sem2):
        cid = lax.axis_index("c")
        sid = lax.axis_index("s")
        wid = cid * NS + sid

        pltpu.sync_copy(sig_hbm, sig_v)
        zero16 = jnp.zeros((L,), jnp.float32)

        def zb(i, c):
            for k in range(kd):
                crows_v[i, pl.ds(k * L, L)] = zero16
            for k in range(8):
                vbuf_v[i, pl.ds(k * L, L)] = zero16
            return c
        lax.fori_loop(0, block_e, zb, 0)

        for j in range(chunk_iters):
            ch = sid + NS * j

            @pl.when(ch < nchunk)
            def _():
                rr = ch * chunk_rows
                pltpu.sync_copy(crows_v, s_sh.at[pl.ds(rr, chunk_rows)])
        pltpu.sync_copy(vbuf_v, deg_sh.at[pl.ds(sid * dpt, dpt)])
        plsc.subcore_barrier()

        sig = sig_v[pl.ds(0, L)]
        e0 = wid * ept

        def blk_body(b, c):
            base = e0 + b * block_e
            pltpu.sync_copy(row_hbm.at[pl.ds(base, block_e)], ridx_v)
            pltpu.sync_copy(col_hbm.at[pl.ds(base, block_e)], cidx_v)
            cp1 = pltpu.async_copy(x_hbm.at[ridx_v], rrows_v, sem1)
            cp2 = pltpu.async_copy(x_hbm.at[cidx_v], crows_v, sem2)
            cp1.wait()
            cp2.wait()

            lanes = lax.iota(jnp.int32, L)
            dnums = lax.GatherDimensionNumbers(
                offset_dims=(), collapsed_slice_dims=(0,), start_index_map=(0,))

            def group_body(g, cc):
                g0 = g * L
                rv = ridx_v[pl.ds(g0, L)]
                didx_v[pl.ds(g0, L)] = lax.shift_right_logical(rv, 3)
                for j in range(L):
                    i = g0 + j
                    xcs = []
                    acc = None
                    for k in range(kd):
                        xr = rrows_v[i, pl.ds(k * L, L)]
                        xc = crows_v[i, pl.ds(k * L, L)]
                        xcs.append(xc)
                        df = xr - xc
                        sq = df * df
                        acc = sq if acc is None else acc + sq
                    for m in (8, 4, 2, 1):
                        acc = acc + lax.gather(
                            acc, (lanes ^ m)[:, None], dnums, slice_sizes=(1,),
                            mode=lax.GatherScatterMode.PROMISE_IN_BOUNDS)
                    vv = jnp.exp(sig * acc)
                    sub = rv[j] & 7
                    for k in range(8):
                        hit = (1 - jnp.minimum(sub ^ k, 1)).astype(jnp.float32)
                        vbuf_v[i, pl.ds(k * L, L)] = vv * hit
                    for k in range(kd):
                        crows_v[i, pl.ds(k * L, L)] = xcs[k] * vv
                return cc
            lax.fori_loop(0, block_e // L, group_body, 0)

            pltpu.sync_copy(crows_v, s_sh.at[ridx_v], add=True)
            pltpu.sync_copy(vbuf_v, deg_sh.at[didx_v], add=True)
            return c
        lax.fori_loop(0, nblk, blk_body, 0)
        plsc.subcore_barrier()

        for j in range(chunk_iters):
            ch = sid + NS * j

            @pl.when(ch < nchunk)
            def _():
                rr = ch * chunk_rows
                pltpu.sync_copy(s_sh.at[pl.ds(rr, chunk_rows)], crows_v)
                pltpu.sync_copy(crows_v, s_out.at[cid, pl.ds(rr, chunk_rows)])
        pltpu.sync_copy(deg_sh.at[pl.ds(sid * dpt, dpt)], vbuf_v)
        pltpu.sync_copy(vbuf_v, deg_out.at[cid, pl.ds(sid * dpt, dpt)])

    return step


def _normalize(s2, deg2, blk=400):
    n, d = s2.shape[1], s2.shape[2]

    def body(s_ref, deg_ref, o_ref):
        s = s_ref[0] + s_ref[1]
        deg = deg_ref[0, :, 0:1] + deg_ref[1, :, 0:1]
        inv = jnp.where(deg > 0.0, 1.0 / deg, 0.0)
        o_ref[...] = s * inv

    return pl.pallas_call(
        body,
        grid=(n // blk,),
        in_specs=[
            pl.BlockSpec((NC, blk, d), lambda i: (0, i, 0)),
            pl.BlockSpec((NC, blk, L), lambda i: (0, i, 0)),
        ],
        out_specs=pl.BlockSpec((blk, d), lambda i: (i, 0)),
        out_shape=jax.ShapeDtypeStruct((n, d), jnp.float32),
    )(s2, deg2)


def _project(x0, x1, x2, y1, y2, W, blk=400):
    n, d = x0.shape
    c = y1.shape[1]
    dims = 3 * d + 2 * c

    def body(x0_ref, x1_ref, x2_ref, y1_ref, y2_ref, w_ref, o_ref):
        hp = jax.lax.Precision.HIGHEST
        acc = jnp.dot(x0_ref[...], w_ref[0:d, :], precision=hp)
        acc += jnp.dot(x1_ref[...], w_ref[d:2 * d, :], precision=hp)
        acc += jnp.dot(x2_ref[...], w_ref[2 * d:3 * d, :], precision=hp)
        acc += jnp.dot(y1_ref[...], w_ref[3 * d:3 * d + c, :], precision=hp)
        acc += jnp.dot(y2_ref[...], w_ref[3 * d + c:dims, :], precision=hp)
        o_ref[...] = acc

    cdim = W.shape[1]
    return pl.pallas_call(
        body,
        grid=(n // blk,),
        in_specs=[
            pl.BlockSpec((blk, d), lambda i: (i, 0)),
            pl.BlockSpec((blk, d), lambda i: (i, 0)),
            pl.BlockSpec((blk, d), lambda i: (i, 0)),
            pl.BlockSpec((blk, c), lambda i: (i, 0)),
            pl.BlockSpec((blk, c), lambda i: (i, 0)),
            pl.BlockSpec((dims, cdim), lambda i: (0, 0)),
        ],
        out_specs=pl.BlockSpec((blk, cdim), lambda i: (i, 0)),
        out_shape=jax.ShapeDtypeStruct((n, cdim), jnp.float32),
    )(x0, x1, x2, y1, y2, W)


def kernel(x, y_one_hot_train, W, sigmas, row, col):
    n, d = x.shape
    e = row.shape[0]
    c = y_one_hot_train.shape[1]

    nis2 = -1.0 / (sigmas * sigmas)

    step = _make_step(n, e, d)

    def run_step(cur, it):
        s2, deg2 = step(cur, row, col, jnp.full((128,), nis2[it], jnp.float32))
        deg2 = deg2.reshape(NC, -1, L)[:, :n, :]
        return _normalize(s2, deg2)

    x_outs = [x]
    cur = x
    for it in range(2):
        cur = run_step(cur, it)
        x_outs.append(cur)

    y_outs = []
    cur = jnp.concatenate(
        [y_one_hot_train, jnp.zeros((n, d - c), jnp.float32)], axis=1)
    for it in range(2):
        cur = run_step(cur, 2 + it)
        y_outs.append(cur)

    return _project(x_outs[0], x_outs[1], x_outs[2],
                    y_outs[0][:, :c], y_outs[1][:, :c], W)


# speedup vs baseline: 5.1405x; 1.1775x over previous
import functools

import jax
import jax.numpy as jnp
from jax import lax
from jax.experimental import pallas as pl
from jax.experimental.pallas import tpu as pltpu
from jax.experimental.pallas import tpu_sc as plsc

NC = 2
NS = 16
L = 16
BE = 16
SETS = 3


def _lgather(vec, idx):
    dnums = lax.GatherDimensionNumbers(
        offset_dims=(), collapsed_slice_dims=(0,), start_index_map=(0,))
    return lax.gather(vec, idx[:, None], dnums, slice_sizes=(1,),
                      mode=lax.GatherScatterMode.PROMISE_IN_BOUNDS)


def _make_step(n, e, d, chunk_rows=80):
    ept = e // (NC * NS)
    nblk = ept // BE
    assert ept % BE == 0 and nblk > 8
    assert n % chunk_rows == 0 and chunk_rows % 8 == 0
    nchunk = n // chunk_rows
    chunk_iters = -(-nchunk // NS)
    kd = d // L
    npad = -(-n // 1024) * 1024
    nslot = npad // 8
    dpt = nslot // NS
    assert dpt == chunk_rows

    mesh = plsc.VectorSubcoreMesh(
        core_axis_name="c", subcore_axis_name="s", num_cores=NC, num_subcores=NS
    )

    scratch = []
    for _ in range(SETS):
        scratch += [
            pltpu.VMEM((2 * BE,), jnp.int32),
            pltpu.VMEM((BE,), jnp.int32),
            pltpu.VMEM((BE,), jnp.int32),
            pltpu.VMEM((BE,), jnp.int32),
            pltpu.VMEM((BE, d), jnp.float32),
            pltpu.VMEM((BE, d), jnp.float32),
            pltpu.VMEM((BE, 128), jnp.float32),
        ]
    scratch += [
        pltpu.VMEM((chunk_rows, 128), jnp.float32),
        pltpu.VMEM((128,), jnp.float32),
        pltpu.VMEM_SHARED((n, d), jnp.float32),
        pltpu.VMEM_SHARED((nslot, 128), jnp.float32),
    ]
    scratch += [pltpu.SemaphoreType.DMA] * (3 * SETS)

    @functools.partial(
        pl.kernel,
        out_type=(
            jax.ShapeDtypeStruct((NC, n, d), jnp.float32),
            jax.ShapeDtypeStruct((NC, nslot, 128), jnp.float32),
        ),
        mesh=mesh,
        scratch_types=scratch,
    )
    def step(x_hbm, rc_hbm, sig_hbm, s_out, deg_out, *refs):
        sets = [refs[7 * s:7 * s + 7] for s in range(SETS)]
        bounce_v, sig_v, s_sh, deg_sh = refs[7 * SETS:7 * SETS + 4]
        sems = refs[7 * SETS + 4:]
        isem = sems[0:SETS]
        gsem = sems[SETS:2 * SETS]
        ssem = sems[2 * SETS:3 * SETS]

        cid = lax.axis_index("c")
        sid = lax.axis_index("s")
        wid = cid * NS + sid
        e0 = wid * ept

        pltpu.sync_copy(sig_hbm, sig_v)
        zero16 = jnp.zeros((L,), jnp.float32)

        def zb(i, c):
            for k in range(8):
                bounce_v[i, pl.ds(k * L, L)] = zero16
            return c
        lax.fori_loop(0, chunk_rows, zb, 0)

        for j in range(chunk_iters):
            ch = sid + NS * j

            @pl.when(ch < nchunk)
            def _():
                pltpu.sync_copy(
                    bounce_v, s_sh.at[pl.ds(ch * chunk_rows, chunk_rows)])
        pltpu.sync_copy(bounce_v, deg_sh.at[pl.ds(sid * dpt, dpt)])
        plsc.subcore_barrier()

        sig = sig_v[pl.ds(0, L)]
        lanes = lax.iota(jnp.int32, L)
        idx_e = (2 * lanes) & 15
        idx_o = (2 * lanes + 1) & 15
        mh = lax.shift_right_logical(lanes, 3)
        ml = 1 - mh

        def rc_ofs(b):
            return 2 * e0 + 2 * BE * b

        def issue_idx(b, s):
            pltpu.async_copy(rc_hbm.at[pl.ds(rc_ofs(b), 2 * BE)],
                             sets[s][0], isem[s])

        def wait_idx(b, s):
            pltpu.make_async_copy(rc_hbm.at[pl.ds(rc_ofs(b), 2 * BE)],
                                  sets[s][0], isem[s]).wait()

        def deint(s):
            rc_v, ridx_v, cidx_v, didx_v = sets[s][:4]
            v0 = rc_v[pl.ds(0, L)]
            v1 = rc_v[pl.ds(L, L)]
            r = _lgather(v0, idx_e) * ml + _lgather(v1, idx_e) * mh
            c = _lgather(v0, idx_o) * ml + _lgather(v1, idx_o) * mh
            ridx_v[...] = r
            cidx_v[...] = c
            didx_v[...] = lax.shift_right_logical(r, 3)

        def issue_gathers(s):
            _, ridx_v, cidx_v, _, rrows_v, crows_v, _ = sets[s]
            pltpu.async_copy(x_hbm.at[ridx_v], rrows_v, gsem[s])
            pltpu.async_copy(x_hbm.at[cidx_v], crows_v, gsem[s])

        def wait_gathers(s):
            _, ridx_v, cidx_v, _, rrows_v, crows_v, _ = sets[s]
            pltpu.make_async_copy(x_hbm.at[ridx_v], rrows_v, gsem[s]).wait()
            pltpu.make_async_copy(x_hbm.at[cidx_v], crows_v, gsem[s]).wait()

        def issue_scatters(s):
            _, ridx_v, _, didx_v, _, crows_v, vbuf_v = sets[s]
            pltpu.async_copy(crows_v, s_sh.at[ridx_v], ssem[s], add=True)
            pltpu.async_copy(vbuf_v, deg_sh.at[didx_v], ssem[s], add=True)

        def wait_scatters(s):
            _, ridx_v, _, didx_v, _, crows_v, vbuf_v = sets[s]
            pltpu.make_async_copy(crows_v, s_sh.at[ridx_v], ssem[s]).wait()
            pltpu.make_async_copy(vbuf_v, deg_sh.at[didx_v], ssem[s]).wait()

        def compute16(s):
            _, ridx_v, _, _, rrows_v, crows_v, vbuf_v = sets[s]
            rv = ridx_v[...]
            for j in range(BE):
                xcs = []
                acc = None
                for k in range(kd):
                    xr = rrows_v[j, pl.ds(k * L, L)]
                    xc = crows_v[j, pl.ds(k * L, L)]
                    xcs.append(xc)
                    df = xr - xc
                    sq = df * df
                    acc = sq if acc is None else acc + sq
                for m in (8, 4, 2, 1):
                    acc = acc + _lgather(acc, lanes ^ m)
                vv = jnp.exp(sig * acc)
                sub = rv[j] & 7
                for k in range(8):
                    hit = (1 - jnp.minimum(sub ^ k, 1)).astype(jnp.float32)
                    vbuf_v[j, pl.ds(k * L, L)] = vv * hit
                for k in range(kd):
                    crows_v[j, pl.ds(k * L, L)] = xcs[k] * vv

        def proc(b, s, first=False, pf_gather=True, pf_idx=True):
            wait_gathers(s)
            compute16(s)
            issue_scatters(s)
            if pf_gather:
                s2 = (s + 2) % SETS
                if not first:
                    wait_scatters(s2)
                wait_idx(b + 2, s2)
                deint(s2)
                issue_gathers(s2)
            if pf_idx:
                issue_idx(b + 3, s)

        for s in range(SETS):
            issue_idx(s, s)
        for s in range(2):
            wait_idx(s, s)
            deint(s)
            issue_gathers(s)
        proc(0, 0, first=True)
        proc(1, 1)
        proc(2, 2)

        hi = (nblk - 2) // 3

        def body(t, c):
            b0 = 3 * t
            proc(b0, 0)
            proc(b0 + 1, 1)
            proc(b0 + 2, 2)
            return c
        lax.fori_loop(1, hi, body, 0)

        for b in range(3 * hi, nblk):
            proc(b, b % SETS, pf_gather=(b + 2 < nblk), pf_idx=(b + 3 < nblk))
        for b in range(nblk - 3, nblk):
            wait_scatters(b % SETS)
        plsc.subcore_barrier()

        for j in range(chunk_iters):
            ch = sid + NS * j

            @pl.when(ch < nchunk)
            def _():
                rr = ch * chunk_rows
                pltpu.sync_copy(s_sh.at[pl.ds(rr, chunk_rows)], bounce_v)
                pltpu.sync_copy(bounce_v, s_out.at[cid, pl.ds(rr, chunk_rows)])
        pltpu.sync_copy(deg_sh.at[pl.ds(sid * dpt, dpt)], bounce_v)
        pltpu.sync_copy(bounce_v, deg_out.at[cid, pl.ds(sid * dpt, dpt)])

    return step


def _normalize(s2, deg2, blk=400):
    n, d = s2.shape[1], s2.shape[2]

    def body(s_ref, deg_ref, o_ref):
        s = s_ref[0] + s_ref[1]
        deg = deg_ref[0, :, 0:1] + deg_ref[1, :, 0:1]
        inv = jnp.where(deg > 0.0, 1.0 / deg, 0.0)
        o_ref[...] = s * inv

    return pl.pallas_call(
        body,
        grid=(n // blk,),
        in_specs=[
            pl.BlockSpec((NC, blk, d), lambda i: (0, i, 0)),
            pl.BlockSpec((NC, blk, L), lambda i: (0, i, 0)),
        ],
        out_specs=pl.BlockSpec((blk, d), lambda i: (i, 0)),
        out_shape=jax.ShapeDtypeStruct((n, d), jnp.float32),
    )(s2, deg2)


def _project(x0, x1, x2, y1, y2, W, blk=400):
    n, d = x0.shape
    c = y1.shape[1]
    dims = 3 * d + 2 * c

    def body(x0_ref, x1_ref, x2_ref, y1_ref, y2_ref, w_ref, o_ref):
        hp = jax.lax.Precision.HIGHEST
        acc = jnp.dot(x0_ref[...], w_ref[0:d, :], precision=hp)
        acc += jnp.dot(x1_ref[...], w_ref[d:2 * d, :], precision=hp)
        acc += jnp.dot(x2_ref[...], w_ref[2 * d:3 * d, :], precision=hp)
        acc += jnp.dot(y1_ref[...], w_ref[3 * d:3 * d + c, :], precision=hp)
        acc += jnp.dot(y2_ref[...], w_ref[3 * d + c:dims, :], precision=hp)
        o_ref[...] = acc

    cdim = W.shape[1]
    return pl.pallas_call(
        body,
        grid=(n // blk,),
        in_specs=[
            pl.BlockSpec((blk, d), lambda i: (i, 0)),
            pl.BlockSpec((blk, d), lambda i: (i, 0)),
            pl.BlockSpec((blk, d), lambda i: (i, 0)),
            pl.BlockSpec((blk, c), lambda i: (i, 0)),
            pl.BlockSpec((blk, c), lambda i: (i, 0)),
            pl.BlockSpec((dims, cdim), lambda i: (0, 0)),
        ],
        out_specs=pl.BlockSpec((blk, cdim), lambda i: (i, 0)),
        out_shape=jax.ShapeDtypeStruct((n, cdim), jnp.float32),
    )(x0, x1, x2, y1, y2, W)


def kernel(x, y_one_hot_train, W, sigmas, row, col):
    n, d = x.shape
    e = row.shape[0]
    c = y_one_hot_train.shape[1]

    nis2 = -1.0 / (sigmas * sigmas)
    rc = jnp.stack([row, col], axis=1).reshape(-1)

    step = _make_step(n, e, d)

    def run_step(cur, it):
        s2, deg2 = step(cur, rc, jnp.full((128,), nis2[it], jnp.float32))
        deg2 = deg2.reshape(NC, -1, L)[:, :n, :]
        return _normalize(s2, deg2)

    x_outs = [x]
    cur = x
    for it in range(2):
        cur = run_step(cur, it)
        x_outs.append(cur)

    y_outs = []
    cur = jnp.concatenate(
        [y_one_hot_train, jnp.zeros((n, d - c), jnp.float32)], axis=1)
    for it in range(2):
        cur = run_step(cur, 2 + it)
        y_outs.append(cur)

    return _project(x_outs[0], x_outs[1], x_outs[2],
                    y_outs[0][:, :c], y_outs[1][:, :c], W)


# speedup vs baseline: 5.1595x; 1.0037x over previous
import functools

import jax
import jax.numpy as jnp
from jax import lax
from jax.experimental import pallas as pl
from jax.experimental.pallas import tpu as pltpu
from jax.experimental.pallas import tpu_sc as plsc

NC = 2
NS = 16
L = 16
BE = 16
SETS = 3


def _lgather(vec, idx):
    dnums = lax.GatherDimensionNumbers(
        offset_dims=(), collapsed_slice_dims=(0,), start_index_map=(0,))
    return lax.gather(vec, idx[:, None], dnums, slice_sizes=(1,),
                      mode=lax.GatherScatterMode.PROMISE_IN_BOUNDS)


def _make_step(n, e, d, chunk_rows=80):
    ept = e // (NC * NS)
    nblk = ept // BE
    assert ept % BE == 0 and nblk > 8
    assert n % chunk_rows == 0 and chunk_rows % 8 == 0
    nchunk = n // chunk_rows
    chunk_iters = -(-nchunk // NS)
    kd = d // L
    npad = -(-n // 1024) * 1024
    nslot = npad // 8
    dpt = nslot // NS
    assert dpt == chunk_rows

    mesh = plsc.VectorSubcoreMesh(
        core_axis_name="c", subcore_axis_name="s", num_cores=NC, num_subcores=NS
    )

    scratch = []
    for _ in range(SETS):
        scratch += [
            pltpu.VMEM((2 * BE,), jnp.int32),
            pltpu.VMEM((BE,), jnp.int32),
            pltpu.VMEM((BE,), jnp.int32),
            pltpu.VMEM((2 * BE, d), jnp.float32),
            pltpu.VMEM((BE, d), jnp.float32),
            pltpu.VMEM((BE, 128), jnp.float32),
        ]
    scratch += [
        pltpu.VMEM((chunk_rows, 128), jnp.float32),
        pltpu.VMEM((128,), jnp.float32),
        pltpu.VMEM_SHARED((n, d), jnp.float32),
        pltpu.VMEM_SHARED((nslot, 128), jnp.float32),
    ]
    scratch += [pltpu.SemaphoreType.DMA] * (3 * SETS)

    @functools.partial(
        pl.kernel,
        out_type=(
            jax.ShapeDtypeStruct((NC, n, d), jnp.float32),
            jax.ShapeDtypeStruct((NC, nslot, 128), jnp.float32),
        ),
        mesh=mesh,
        scratch_types=scratch,
    )
    def step(x_hbm, rc_hbm, sig_hbm, s_out, deg_out, *refs):
        sets = [refs[6 * s:6 * s + 6] for s in range(SETS)]
        bounce_v, sig_v, s_sh, deg_sh = refs[6 * SETS:6 * SETS + 4]
        sems = refs[6 * SETS + 4:]
        isem = sems[0:SETS]
        gsem = sems[SETS:2 * SETS]
        ssem = sems[2 * SETS:3 * SETS]

        cid = lax.axis_index("c")
        sid = lax.axis_index("s")
        wid = cid * NS + sid
        e0 = wid * ept

        pltpu.sync_copy(sig_hbm, sig_v)
        zero16 = jnp.zeros((L,), jnp.float32)

        def zb(i, c):
            for k in range(8):
                bounce_v[i, pl.ds(k * L, L)] = zero16
            return c
        lax.fori_loop(0, chunk_rows, zb, 0)

        for j in range(chunk_iters):
            ch = sid + NS * j

            @pl.when(ch < nchunk)
            def _():
                pltpu.sync_copy(
                    bounce_v, s_sh.at[pl.ds(ch * chunk_rows, chunk_rows)])
        pltpu.sync_copy(bounce_v, deg_sh.at[pl.ds(sid * dpt, dpt)])
        plsc.subcore_barrier()

        sig = sig_v[pl.ds(0, L)]
        lanes = lax.iota(jnp.int32, L)
        idx_e = (2 * lanes) & 15
        idx_o = (2 * lanes + 1) & 15
        mh = lax.shift_right_logical(lanes, 3)
        ml = 1 - mh

        def rc_ofs(b):
            return 2 * e0 + 2 * BE * b

        def issue_idx(b, s):
            pltpu.async_copy(rc_hbm.at[pl.ds(rc_ofs(b), 2 * BE)],
                             sets[s][0], isem[s])

        def wait_idx(b, s):
            pltpu.make_async_copy(rc_hbm.at[pl.ds(rc_ofs(b), 2 * BE)],
                                  sets[s][0], isem[s]).wait()

        def deint(s):
            rc_v, ridx_v, didx_v = sets[s][:3]
            v0 = rc_v[pl.ds(0, L)]
            v1 = rc_v[pl.ds(L, L)]
            r = _lgather(v0, idx_e) * ml + _lgather(v1, idx_e) * mh
            ridx_v[...] = r
            didx_v[...] = lax.shift_right_logical(r, 3)

        def issue_gathers(s):
            rc_v, _, _, gbuf_v, _, _ = sets[s]
            pltpu.async_copy(x_hbm.at[rc_v], gbuf_v, gsem[s])

        def wait_gathers(s):
            rc_v, _, _, gbuf_v, _, _ = sets[s]
            pltpu.make_async_copy(x_hbm.at[rc_v], gbuf_v, gsem[s]).wait()

        def issue_scatters(s):
            _, ridx_v, didx_v, _, cbuf_v, vbuf_v = sets[s]
            pltpu.async_copy(cbuf_v, s_sh.at[ridx_v], ssem[s], add=True)
            pltpu.async_copy(vbuf_v, deg_sh.at[didx_v], ssem[s], add=True)

        def wait_scatters(s):
            _, ridx_v, didx_v, _, cbuf_v, vbuf_v = sets[s]
            pltpu.make_async_copy(cbuf_v, s_sh.at[ridx_v], ssem[s]).wait()
            pltpu.make_async_copy(vbuf_v, deg_sh.at[didx_v], ssem[s]).wait()

        def compute16(s):
            _, ridx_v, _, gbuf_v, cbuf_v, vbuf_v = sets[s]
            rv = ridx_v[...]
            for j in range(BE):
                xcs = []
                acc = None
                for k in range(kd):
                    xr = gbuf_v[2 * j, pl.ds(k * L, L)]
                    xc = gbuf_v[2 * j + 1, pl.ds(k * L, L)]
                    xcs.append(xc)
                    df = xr - xc
                    sq = df * df
                    acc = sq if acc is None else acc + sq
                for m in (8, 4, 2, 1):
                    acc = acc + _lgather(acc, lanes ^ m)
                vv = jnp.exp(sig * acc)
                sub = rv[j] & 7
                for k in range(8):
                    hit = (1 - jnp.minimum(sub ^ k, 1)).astype(jnp.float32)
                    vbuf_v[j, pl.ds(k * L, L)] = vv * hit
                for k in range(kd):
                    cbuf_v[j, pl.ds(k * L, L)] = xcs[k] * vv

        def proc(b, s, first=False, pf_gather=True, pf_idx=True):
            wait_gathers(s)
            compute16(s)
            issue_scatters(s)
            if pf_gather:
                s2 = (s + 2) % SETS
                if not first:
                    wait_scatters(s2)
                wait_idx(b + 2, s2)
                deint(s2)
                issue_gathers(s2)
            if pf_idx:
                issue_idx(b + 3, s)

        for s in range(SETS):
            issue_idx(s, s)
        for s in range(2):
            wait_idx(s, s)
            deint(s)
            issue_gathers(s)
        proc(0, 0, first=True)
        proc(1, 1)
        proc(2, 2)

        hi = (nblk - 2) // 3

        def body(t, c):
            b0 = 3 * t
            proc(b0, 0)
            proc(b0 + 1, 1)
            proc(b0 + 2, 2)
            return c
        lax.fori_loop(1, hi, body, 0)

        for b in range(3 * hi, nblk):
            proc(b, b % SETS, pf_gather=(b + 2 < nblk), pf_idx=(b + 3 < nblk))
        for b in range(nblk - 3, nblk):
            wait_scatters(b % SETS)
        plsc.subcore_barrier()

        for j in range(chunk_iters):
            ch = sid + NS * j

            @pl.when(ch < nchunk)
            def _():
                rr = ch * chunk_rows
                pltpu.sync_copy(s_sh.at[pl.ds(rr, chunk_rows)], bounce_v)
                pltpu.sync_copy(bounce_v, s_out.at[cid, pl.ds(rr, chunk_rows)])
        pltpu.sync_copy(deg_sh.at[pl.ds(sid * dpt, dpt)], bounce_v)
        pltpu.sync_copy(bounce_v, deg_out.at[cid, pl.ds(sid * dpt, dpt)])

    return step


def _normalize(s2, deg2, blk=400):
    n, d = s2.shape[1], s2.shape[2]

    def body(s_ref, deg_ref, o_ref):
        s = s_ref[0] + s_ref[1]
        deg = deg_ref[0, :, 0:1] + deg_ref[1, :, 0:1]
        inv = jnp.where(deg > 0.0, 1.0 / deg, 0.0)
        o_ref[...] = s * inv

    return pl.pallas_call(
        body,
        grid=(n // blk,),
        in_specs=[
            pl.BlockSpec((NC, blk, d), lambda i: (0, i, 0)),
            pl.BlockSpec((NC, blk, L), lambda i: (0, i, 0)),
        ],
        out_specs=pl.BlockSpec((blk, d), lambda i: (i, 0)),
        out_shape=jax.ShapeDtypeStruct((n, d), jnp.float32),
    )(s2, deg2)


def _project(x0, x1, x2, y1, y2, W, blk=400):
    n, d = x0.shape
    c = y1.shape[1]
    dims = 3 * d + 2 * c

    def body(x0_ref, x1_ref, x2_ref, y1_ref, y2_ref, w_ref, o_ref):
        hp = jax.lax.Precision.HIGHEST
        acc = jnp.dot(x0_ref[...], w_ref[0:d, :], precision=hp)
        acc += jnp.dot(x1_ref[...], w_ref[d:2 * d, :], precision=hp)
        acc += jnp.dot(x2_ref[...], w_ref[2 * d:3 * d, :], precision=hp)
        acc += jnp.dot(y1_ref[...], w_ref[3 * d:3 * d + c, :], precision=hp)
        acc += jnp.dot(y2_ref[...], w_ref[3 * d + c:dims, :], precision=hp)
        o_ref[...] = acc

    cdim = W.shape[1]
    return pl.pallas_call(
        body,
        grid=(n // blk,),
        in_specs=[
            pl.BlockSpec((blk, d), lambda i: (i, 0)),
            pl.BlockSpec((blk, d), lambda i: (i, 0)),
            pl.BlockSpec((blk, d), lambda i: (i, 0)),
            pl.BlockSpec((blk, c), lambda i: (i, 0)),
            pl.BlockSpec((blk, c), lambda i: (i, 0)),
            pl.BlockSpec((dims, cdim), lambda i: (0, 0)),
        ],
        out_specs=pl.BlockSpec((blk, cdim), lambda i: (i, 0)),
        out_shape=jax.ShapeDtypeStruct((n, cdim), jnp.float32),
    )(x0, x1, x2, y1, y2, W)


def kernel(x, y_one_hot_train, W, sigmas, row, col):
    n, d = x.shape
    e = row.shape[0]
    c = y_one_hot_train.shape[1]

    nis2 = -1.0 / (sigmas * sigmas)
    rc = jnp.stack([row, col], axis=1).reshape(-1)

    step = _make_step(n, e, d)

    def run_step(cur, it):
        s2, deg2 = step(cur, rc, jnp.full((128,), nis2[it], jnp.float32))
        deg2 = deg2.reshape(NC, -1, L)[:, :n, :]
        return _normalize(s2, deg2)

    x_outs = [x]
    cur = x
    for it in range(2):
        cur = run_step(cur, it)
        x_outs.append(cur)

    y_outs = []
    cur = jnp.concatenate(
        [y_one_hot_train, jnp.zeros((n, d - c), jnp.float32)], axis=1)
    for it in range(2):
        cur = run_step(cur, 2 + it)
        y_outs.append(cur)

    return _project(x_outs[0], x_outs[1], x_outs[2],
                    y_outs[0][:, :c], y_outs[1][:, :c], W)


# speedup vs baseline: 5.1847x; 1.0049x over previous
import functools

import jax
import jax.numpy as jnp
from jax import lax
from jax.experimental import pallas as pl
from jax.experimental.pallas import tpu as pltpu
from jax.experimental.pallas import tpu_sc as plsc

NC = 2
NS = 16
L = 16
BE = 16
SETS = 3


def _lgather(vec, idx):
    dnums = lax.GatherDimensionNumbers(
        offset_dims=(), collapsed_slice_dims=(0,), start_index_map=(0,))
    return lax.gather(vec, idx[:, None], dnums, slice_sizes=(1,),
                      mode=lax.GatherScatterMode.PROMISE_IN_BOUNDS)


def _make_step(n, e, d, chunk_rows=80):
    ept = e // (NC * NS)
    nblk = ept // BE
    assert ept % BE == 0 and nblk > 8
    assert n % chunk_rows == 0 and chunk_rows % 8 == 0
    nchunk = n // chunk_rows
    chunk_iters = -(-nchunk // NS)
    kd = d // L
    npad = -(-n // 1024) * 1024
    nslot = npad // 8
    dpt = nslot // NS
    assert dpt == chunk_rows

    mesh = plsc.VectorSubcoreMesh(
        core_axis_name="c", subcore_axis_name="s", num_cores=NC, num_subcores=NS
    )

    scratch = []
    for _ in range(SETS):
        scratch += [
            pltpu.VMEM((2 * BE,), jnp.int32),
            pltpu.VMEM((BE,), jnp.int32),
            pltpu.VMEM((BE,), jnp.int32),
            pltpu.VMEM((BE,), jnp.int32),
            pltpu.VMEM((2 * BE, d), jnp.float32),
            pltpu.VMEM((BE, d), jnp.float32),
            pltpu.VMEM((BE, 128), jnp.float32),
        ]
    scratch += [
        pltpu.VMEM((chunk_rows, 128), jnp.float32),
        pltpu.VMEM((128,), jnp.float32),
        pltpu.VMEM_SHARED((n, d), jnp.float32),
        pltpu.VMEM_SHARED((nslot, 128), jnp.float32),
    ]
    scratch += [pltpu.SemaphoreType.DMA] * (3 * SETS)

    @functools.partial(
        pl.kernel,
        out_type=(
            jax.ShapeDtypeStruct((NC, n, d), jnp.float32),
            jax.ShapeDtypeStruct((NC, nslot, 128), jnp.float32),
        ),
        mesh=mesh,
        scratch_types=scratch,
    )
    def step(x_hbm, rc_hbm, sig_hbm, s_out, deg_out, *refs):
        sets = [refs[7 * s:7 * s + 7] for s in range(SETS)]
        bounce_v, sig_v, s_sh, deg_sh = refs[7 * SETS:7 * SETS + 4]
        sems = refs[7 * SETS + 4:]
        isem = sems[0:SETS]
        gsem = sems[SETS:2 * SETS]
        ssem = sems[2 * SETS:3 * SETS]

        cid = lax.axis_index("c")
        sid = lax.axis_index("s")
        wid = cid * NS + sid
        e0 = wid * ept

        pltpu.sync_copy(sig_hbm, sig_v)
        zero16 = jnp.zeros((L,), jnp.float32)

        def zb(i, c):
            for k in range(8):
                bounce_v[i, pl.ds(k * L, L)] = zero16
            return c
        lax.fori_loop(0, chunk_rows, zb, 0)

        for s in range(SETS):
            oldc_v, vbuf_v = sets[s][3], sets[s][6]
            for j in range(BE):
                for k in range(8):
                    vbuf_v[j, pl.ds(k * L, L)] = zero16
            oldc_v[...] = jnp.zeros((L,), jnp.int32)

        for j in range(chunk_iters):
            ch = sid + NS * j

            @pl.when(ch < nchunk)
            def _():
                pltpu.sync_copy(
                    bounce_v, s_sh.at[pl.ds(ch * chunk_rows, chunk_rows)])
        pltpu.sync_copy(bounce_v, deg_sh.at[pl.ds(sid * dpt, dpt)])
        plsc.subcore_barrier()

        sig = sig_v[pl.ds(0, L)]
        lanes = lax.iota(jnp.int32, L)
        idx_e = (2 * lanes) & 15
        idx_o = (2 * lanes + 1) & 15
        mh = lax.shift_right_logical(lanes, 3)
        ml = 1 - mh

        def rc_ofs(b):
            return 2 * e0 + 2 * BE * b

        def issue_idx(b, s):
            pltpu.async_copy(rc_hbm.at[pl.ds(rc_ofs(b), 2 * BE)],
                             sets[s][0], isem[s])

        def wait_idx(b, s):
            pltpu.make_async_copy(rc_hbm.at[pl.ds(rc_ofs(b), 2 * BE)],
                                  sets[s][0], isem[s]).wait()

        def deint(s):
            rc_v, ridx_v, didx_v = sets[s][:3]
            v0 = rc_v[pl.ds(0, L)]
            v1 = rc_v[pl.ds(L, L)]
            r = _lgather(v0, idx_e) * ml + _lgather(v1, idx_e) * mh
            ridx_v[...] = r
            didx_v[...] = lax.shift_right_logical(r, 3)

        def issue_gathers(s):
            rc_v, gbuf_v = sets[s][0], sets[s][4]
            pltpu.async_copy(x_hbm.at[rc_v], gbuf_v, gsem[s])

        def wait_gathers(s):
            rc_v, gbuf_v = sets[s][0], sets[s][4]
            pltpu.make_async_copy(x_hbm.at[rc_v], gbuf_v, gsem[s]).wait()

        def issue_scatters(s):
            _, ridx_v, didx_v, _, _, cbuf_v, vbuf_v = sets[s]
            pltpu.async_copy(cbuf_v, s_sh.at[ridx_v], ssem[s], add=True)
            pltpu.async_copy(vbuf_v, deg_sh.at[didx_v], ssem[s], add=True)

        def wait_scatters(s):
            _, ridx_v, didx_v, _, _, cbuf_v, vbuf_v = sets[s]
            pltpu.make_async_copy(cbuf_v, s_sh.at[ridx_v], ssem[s]).wait()
            pltpu.make_async_copy(vbuf_v, deg_sh.at[didx_v], ssem[s]).wait()

        def compute16(s):
            _, ridx_v, _, oldc_v, gbuf_v, cbuf_v, vbuf_v = sets[s]
            rv = ridx_v[...]
            vvv = zero16
            for j in range(BE):
                xcs = []
                sqs = []
                for k in range(kd):
                    xr = gbuf_v[2 * j, pl.ds(k * L, L)]
                    xc = gbuf_v[2 * j + 1, pl.ds(k * L, L)]
                    xcs.append(xc)
                    df = xr - xc
                    sqs.append(df * df)
                while len(sqs) > 1:
                    sqs = [a + b for a, b in zip(sqs[0::2], sqs[1::2])]
                acc = sqs[0]
                for m in (8, 4, 2, 1):
                    acc = acc + _lgather(acc, lanes ^ m)
                vv = jnp.exp(sig * acc)
                sub = rv[j] & 7
                for k in range(8):
                    hit = (1 - jnp.minimum(sub ^ k, 1)).astype(jnp.float32)
                    vbuf_v[j, pl.ds(k * L, L)] = vv * hit
                for k in range(kd):
                    cbuf_v[j, pl.ds(k * L, L)] = xcs[k] * vv

        def proc(b, s, first=False, pf_gather=True, pf_idx=True):
            wait_gathers(s)
            compute16(s)
            issue_scatters(s)
            if pf_gather:
                s2 = (s + 2) % SETS
                if not first:
                    wait_scatters(s2)
                wait_idx(b + 2, s2)
                deint(s2)
                issue_gathers(s2)
            if pf_idx:
                issue_idx(b + 3, s)

        for s in range(SETS):
            issue_idx(s, s)
        for s in range(2):
            wait_idx(s, s)
            deint(s)
            issue_gathers(s)
        proc(0, 0, first=True)
        proc(1, 1)
        proc(2, 2)

        hi = (nblk - 2) // 3

        def body(t, c):
            b0 = 3 * t
            proc(b0, 0)
            proc(b0 + 1, 1)
            proc(b0 + 2, 2)
            return c
        lax.fori_loop(1, hi, body, 0)

        for b in range(3 * hi, nblk):
            proc(b, b % SETS, pf_gather=(b + 2 < nblk), pf_idx=(b + 3 < nblk))
        for b in range(nblk - 3, nblk):
            wait_scatters(b % SETS)
        plsc.subcore_barrier()

        for j in range(chunk_iters):
            ch = sid + NS * j

            @pl.when(ch < nchunk)
            def _():
                rr = ch * chunk_rows
                pltpu.sync_copy(s_sh.at[pl.ds(rr, chunk_rows)], bounce_v)
                pltpu.sync_copy(bounce_v, s_out.at[cid, pl.ds(rr, chunk_rows)])
        pltpu.sync_copy(deg_sh.at[pl.ds(sid * dpt, dpt)], bounce_v)
        pltpu.sync_copy(bounce_v, deg_out.at[cid, pl.ds(sid * dpt, dpt)])

    return step


def _normalize(s2, deg2, blk=400):
    n, d = s2.shape[1], s2.shape[2]

    def body(s_ref, deg_ref, o_ref):
        s = s_ref[0] + s_ref[1]
        deg = deg_ref[0, :, 0:1] + deg_ref[1, :, 0:1]
        inv = jnp.where(deg > 0.0, 1.0 / deg, 0.0)
        o_ref[...] = s * inv

    return pl.pallas_call(
        body,
        grid=(n // blk,),
        in_specs=[
            pl.BlockSpec((NC, blk, d), lambda i: (0, i, 0)),
            pl.BlockSpec((NC, blk, L), lambda i: (0, i, 0)),
        ],
        out_specs=pl.BlockSpec((blk, d), lambda i: (i, 0)),
        out_shape=jax.ShapeDtypeStruct((n, d), jnp.float32),
    )(s2, deg2)


def _project(x0, x1, x2, y1, y2, W, blk=400):
    n, d = x0.shape
    c = y1.shape[1]
    dims = 3 * d + 2 * c

    def body(x0_ref, x1_ref, x2_ref, y1_ref, y2_ref, w_ref, o_ref):
        hp = jax.lax.Precision.HIGHEST
        acc = jnp.dot(x0_ref[...], w_ref[0:d, :], precision=hp)
        acc += jnp.dot(x1_ref[...], w_ref[d:2 * d, :], precision=hp)
        acc += jnp.dot(x2_ref[...], w_ref[2 * d:3 * d, :], precision=hp)
        acc += jnp.dot(y1_ref[...], w_ref[3 * d:3 * d + c, :], precision=hp)
        acc += jnp.dot(y2_ref[...], w_ref[3 * d + c:dims, :], precision=hp)
        o_ref[...] = acc

    cdim = W.shape[1]
    return pl.pallas_call(
        body,
        grid=(n // blk,),
        in_specs=[
            pl.BlockSpec((blk, d), lambda i: (i, 0)),
            pl.BlockSpec((blk, d), lambda i: (i, 0)),
            pl.BlockSpec((blk, d), lambda i: (i, 0)),
            pl.BlockSpec((blk, c), lambda i: (i, 0)),
            pl.BlockSpec((blk, c), lambda i: (i, 0)),
            pl.BlockSpec((dims, cdim), lambda i: (0, 0)),
        ],
        out_specs=pl.BlockSpec((blk, cdim), lambda i: (i, 0)),
        out_shape=jax.ShapeDtypeStruct((n, cdim), jnp.float32),
    )(x0, x1, x2, y1, y2, W)


def kernel(x, y_one_hot_train, W, sigmas, row, col):
    n, d = x.shape
    e = row.shape[0]
    c = y_one_hot_train.shape[1]

    nis2 = -1.0 / (sigmas * sigmas)
    rc = jnp.stack([row, col], axis=1).reshape(-1)

    step = _make_step(n, e, d)

    def run_step(cur, it):
        s2, deg2 = step(cur, rc, jnp.full((128,), nis2[it], jnp.float32))
        deg2 = deg2.reshape(NC, -1, L)[:, :n, :]
        return _normalize(s2, deg2)

    x_outs = [x]
    cur = x
    for it in range(2):
        cur = run_step(cur, it)
        x_outs.append(cur)

    y_outs = []
    cur = jnp.concatenate(
        [y_one_hot_train, jnp.zeros((n, d - c), jnp.float32)], axis=1)
    for it in range(2):
        cur = run_step(cur, 2 + it)
        y_outs.append(cur)

    return _project(x_outs[0], x_outs[1], x_outs[2],
                    y_outs[0][:, :c], y_outs[1][:, :c], W)
